# baseline (speedup 1.0000x reference)
import functools

import jax
import jax.numpy as jnp
from jax import lax
from jax.experimental import pallas as pl
from jax.experimental.pallas import tpu as pltpu
from jax.experimental.pallas import tpu_sc as plsc

N = 10000
E = 320000
D = 128
NE = 2000
EH = 40000

NC = 2
NS = 16
NW = NC * NS
CH = 128
WR = 136

R_G = 10240
R_E = 2048
K_G = 80
K_E = 10


def _zero_vmem(z_v):
    def zrow(i, c):
        def zcol(j, c2):
            z_v[i, pl.ds(j * 16, 16)] = jnp.zeros((16,), jnp.float32)
            return c2
        return lax.fori_loop(0, WR // 16, zcol, c)
    lax.fori_loop(0, CH, zrow, 0)


def _sc_scatter(table, idx, *, R, K):
    rpt = R // NS
    mesh = plsc.VectorSubcoreMesh(core_axis_name="c", subcore_axis_name="s")

    @functools.partial(
        pl.kernel,
        mesh=mesh,
        compiler_params=pltpu.CompilerParams(use_tc_tiling_on_sc=False),
        out_type=jax.ShapeDtypeStruct((NC, R, WR), jnp.float32),
        scratch_types=[
            pltpu.VMEM((2, CH), jnp.int32),
            pltpu.VMEM((2, CH), jnp.int32),
            pltpu.VMEM((CH, WR), jnp.float32),
            pltpu.VMEM((CH, WR), jnp.float32),
            pltpu.VMEM_SHARED((R, WR), jnp.float32),
            pltpu.SemaphoreType.DMA,
            pltpu.SemaphoreType.DMA,
        ],
    )
    def k(table_h, idx_h, out_h, idx0, idx1, rows0, rows1, acc_s, sem0, sem1):
        cid = lax.axis_index("c")
        sid = lax.axis_index("s")
        wid = sid * NC + cid
        base = sid * rpt

        rows = (rows0, rows1)
        idxs = (idx0, idx1)
        sems = (sem0, sem1)

        def fetch_idx(t, b):
            pltpu.sync_copy(idx_h.at[wid, t], idxs[b])

        def issue(b):
            pltpu.async_copy(table_h.at[idxs[b].at[0]], rows[b], sems[b])

        def wait(b):
            pltpu.make_async_copy(table_h.at[idxs[b].at[0]], rows[b],
                                  sems[b]).wait()

        def scatter(b):
            pltpu.sync_copy(rows[b], acc_s.at[idxs[b].at[1]], add=True)

        fetch_idx(0, 0)
        fetch_idx(1, 1)
        _zero_vmem(rows0)

        def zg(r, c):
            pltpu.sync_copy(rows0, acc_s.at[pl.ds(base + r * CH, CH)])
            return c
        lax.fori_loop(0, rpt // CH, zg, 0)

        issue(0)
        issue(1)
        plsc.subcore_barrier()

        @pl.loop(0, K, step=2)
        def _(t):
            wait(0)
            scatter(0)
            fetch_idx(t + 2, 0)
            issue(0)
            wait(1)
            scatter(1)
            fetch_idx(t + 3, 1)
            issue(1)

        wait(0)
        scatter(0)
        wait(1)
        scatter(1)
        plsc.subcore_barrier()

        def wb(r, c):
            pltpu.sync_copy(acc_s.at[pl.ds(base + r * CH, CH)],
                            out_h.at[cid, pl.ds(base + r * CH, CH)])
            return c
        lax.fori_loop(0, rpt // CH, wb, 0)

    return k(table, idx)


def _tc_matmul(Xp, Wz, b144):
    BR = 1024

    def mmk(x_ref, w_ref, b_ref, o_ref):
        o_ref[...] = lax.dot(
            x_ref[...], w_ref[...],
            precision=lax.Precision.HIGHEST,
            preferred_element_type=jnp.float32,
        ) + b_ref[0:1, :]

    return pl.pallas_call(
        mmk,
        grid=(R_G // BR,),
        in_specs=[
            pl.BlockSpec((BR, D), lambda i: (i, 0)),
            pl.BlockSpec((D, WR), lambda i: (0, 0)),
            pl.BlockSpec((8, WR), lambda i: (0, 0)),
        ],
        out_specs=pl.BlockSpec((BR, WR), lambda i: (i, 0)),
        out_shape=jax.ShapeDtypeStruct((R_G, WR), jnp.float32),
    )(Xp, Wz, b144)


def _tc_finalize_y(pe):
    def fk(p_ref, o_ref):
        s = p_ref[0] + p_ref[1]
        o_ref[...] = s / jnp.maximum(s[:, 128:129], 1.0)

    return pl.pallas_call(
        fk,
        out_shape=jax.ShapeDtypeStruct((R_E, WR), jnp.float32),
    )(pe)


def _tc_combine(pg, ph):
    BR = 1024

    def ck(pg_ref, ph_ref, o_ref):
        sg = pg_ref[0] + pg_ref[1]
        sh = ph_ref[0] + ph_ref[1]
        xg = sg[:, :D] / jnp.maximum(sg[:, 128:129], 1.0)
        xh = sh[:, :D] / jnp.maximum(sh[:, 128:129], 1.0)
        o_ref[...] = jnp.maximum(xg * 0.1 + xh * 0.9, 0.0)

    return pl.pallas_call(
        ck,
        grid=(R_G // BR,),
        in_specs=[
            pl.BlockSpec((2, BR, WR), lambda i: (0, i, 0)),
            pl.BlockSpec((2, BR, WR), lambda i: (0, i, 0)),
        ],
        out_specs=pl.BlockSpec((BR, D), lambda i: (i, 0)),
        out_shape=jax.ShapeDtypeStruct((R_G, D), jnp.float32),
    )(pg, ph)


def _pad_pairs(g, s, K, gpad, sbase, R):
    tot = NW * K * CH
    npad = tot - g.shape[0]
    swin = min(CH, R - sbase)
    cyc = jnp.arange(npad, dtype=jnp.int32)
    g = jnp.concatenate([g.astype(jnp.int32), gpad + cyc % CH])
    s = jnp.concatenate([s.astype(jnp.int32), sbase + cyc % swin])
    iv = jnp.stack([g.reshape(NW, K, CH), s.reshape(NW, K, CH)], axis=2)
    lane = jnp.arange(CH, dtype=jnp.int32)
    tail = jnp.stack(
        [jnp.broadcast_to(gpad + lane % CH, (NW, 2, CH)),
         jnp.broadcast_to(sbase + lane % swin, (NW, 2, CH))], axis=2)
    return jnp.concatenate([iv, tail], axis=1)


def kernel(X, edge_index, he_nodes, he_edges, W, b):
    Xp = jnp.pad(X, ((0, R_G - N), (0, 0)))
    Wz = jnp.pad(W.T, ((0, 0), (0, WR - D)))
    b144 = jnp.zeros((8, WR), jnp.float32).at[0, :D].set(b).at[0, D].set(1.0)

    H = _tc_matmul(Xp, Wz, b144)

    iv = _pad_pairs(edge_index[0], edge_index[1], K_G, N, R_G - CH, R_G)
    pg = _sc_scatter(H, iv, R=R_G, K=K_G)

    ie = _pad_pairs(he_nodes, he_edges, K_E, N, NE, R_E)
    pe = _sc_scatter(H, ie, R=R_E, K=K_E)

    Yp = _tc_finalize_y(pe)

    ih = _pad_pairs(he_edges, he_nodes, K_E, R_E - CH, R_G - CH, R_G)
    ph = _sc_scatter(Yp, ih, R=R_G, K=K_E)

    out = _tc_combine(pg, ph)
    return out[:N]


# speedup vs baseline: 1.0347x; 1.0347x over previous
import functools

import jax
import jax.numpy as jnp
from jax import lax
from jax.experimental import pallas as pl
from jax.experimental.pallas import tpu as pltpu
from jax.experimental.pallas import tpu_sc as plsc

N = 10000
E = 320000
D = 128
NE = 2000
EH = 40000

NC = 2
NS = 16
NW = NC * NS
CH = 128
WR = 144

R_G = 10240
R_E = 2048
K_G = 80
K_E = 12


def _zero_vmem(z_v):
    def zrow(i, c):
        def zcol(j, c2):
            z_v[i, pl.ds(j * 16, 16)] = jnp.zeros((16,), jnp.float32)
            return c2
        return lax.fori_loop(0, WR // 16, zcol, c)
    lax.fori_loop(0, CH, zrow, 0)


def _sc_scatter(table, idx, *, R, K):
    rpt = R // NS
    mesh = plsc.VectorSubcoreMesh(core_axis_name="c", subcore_axis_name="s")

    P = K // 2

    @functools.partial(
        pl.kernel,
        mesh=mesh,
        compiler_params=pltpu.CompilerParams(use_tc_tiling_on_sc=False),
        out_type=jax.ShapeDtypeStruct((NC, R, WR), jnp.float32),
        scratch_types=[
            pltpu.VMEM((2, 2, CH), jnp.int32),
            pltpu.VMEM((2, 2, CH), jnp.int32),
            pltpu.VMEM((CH, WR), jnp.float32),
            pltpu.VMEM((CH, WR), jnp.float32),
            pltpu.VMEM_SHARED((R, WR), jnp.float32),
            pltpu.SemaphoreType.DMA,
            pltpu.SemaphoreType.DMA,
        ],
    )
    def k(table_h, idx_h, out_h, pb0, pb1, rows0, rows1, acc_s, sem0, sem1):
        cid = lax.axis_index("c")
        sid = lax.axis_index("s")
        wid = sid * NC + cid
        base = sid * rpt

        rows = (rows0, rows1)
        sems = (sem0, sem1)

        def fetch_pair(p, pb):
            pltpu.sync_copy(idx_h.at[wid, p], pb)

        def issue(pb, j):
            pltpu.async_copy(table_h.at[pb.at[j, 0]], rows[j], sems[j])

        def wait(pb, j):
            pltpu.make_async_copy(table_h.at[pb.at[j, 0]], rows[j],
                                  sems[j]).wait()

        def scatter(pb, j):
            pltpu.sync_copy(rows[j], acc_s.at[pb.at[j, 1]], add=True)

        fetch_pair(0, pb0)
        _zero_vmem(rows0)

        def zg(r, c):
            pltpu.sync_copy(rows0, acc_s.at[pl.ds(base + r * CH, CH)])
            return c
        lax.fori_loop(0, rpt // CH, zg, 0)

        issue(pb0, 0)
        issue(pb0, 1)
        plsc.subcore_barrier()

        @pl.loop(0, P, step=2)
        def _(p):
            fetch_pair(p + 1, pb1)
            wait(pb0, 0)
            scatter(pb0, 0)
            issue(pb1, 0)
            wait(pb0, 1)
            scatter(pb0, 1)
            issue(pb1, 1)
            fetch_pair(p + 2, pb0)
            wait(pb1, 0)
            scatter(pb1, 0)
            issue(pb0, 0)
            wait(pb1, 1)
            scatter(pb1, 1)
            issue(pb0, 1)

        wait(pb0, 0)
        scatter(pb0, 0)
        wait(pb0, 1)
        scatter(pb0, 1)
        plsc.subcore_barrier()

        def wb(r, c):
            pltpu.sync_copy(acc_s.at[pl.ds(base + r * CH, CH)],
                            out_h.at[cid, pl.ds(base + r * CH, CH)])
            return c
        lax.fori_loop(0, rpt // CH, wb, 0)

    return k(table, idx)


def _tc_matmul(Xp, Wz, b144):
    BR = 1024

    def mmk(x_ref, w_ref, b_ref, o_ref):
        o_ref[...] = lax.dot(
            x_ref[...], w_ref[...],
            precision=lax.Precision.HIGHEST,
            preferred_element_type=jnp.float32,
        ) + b_ref[0:1, :]

    return pl.pallas_call(
        mmk,
        grid=(R_G // BR,),
        in_specs=[
            pl.BlockSpec((BR, D), lambda i: (i, 0)),
            pl.BlockSpec((D, WR), lambda i: (0, 0)),
            pl.BlockSpec((8, WR), lambda i: (0, 0)),
        ],
        out_specs=pl.BlockSpec((BR, WR), lambda i: (i, 0)),
        out_shape=jax.ShapeDtypeStruct((R_G, WR), jnp.float32),
    )(Xp, Wz, b144)


def _tc_finalize_y(pe):
    def fk(p_ref, o_ref):
        s = p_ref[0] + p_ref[1]
        o_ref[...] = s / jnp.maximum(s[:, 128:129], 1.0)

    return pl.pallas_call(
        fk,
        out_shape=jax.ShapeDtypeStruct((R_E, WR), jnp.float32),
    )(pe)


def _tc_combine(pg, ph):
    BR = 1024

    def ck(pg_ref, ph_ref, o_ref):
        sg = pg_ref[0] + pg_ref[1]
        sh = ph_ref[0] + ph_ref[1]
        xg = sg[:, :D] / jnp.maximum(sg[:, 128:129], 1.0)
        xh = sh[:, :D] / jnp.maximum(sh[:, 128:129], 1.0)
        o_ref[...] = jnp.maximum(xg * 0.1 + xh * 0.9, 0.0)

    return pl.pallas_call(
        ck,
        grid=(R_G // BR,),
        in_specs=[
            pl.BlockSpec((2, BR, WR), lambda i: (0, i, 0)),
            pl.BlockSpec((2, BR, WR), lambda i: (0, i, 0)),
        ],
        out_specs=pl.BlockSpec((BR, D), lambda i: (i, 0)),
        out_shape=jax.ShapeDtypeStruct((R_G, D), jnp.float32),
    )(pg, ph)


def _pad_pairs(g, s, K, gpad, sbase, R):
    tot = NW * K * CH
    npad = tot - g.shape[0]
    swin = min(CH, R - sbase)
    cyc = jnp.arange(npad, dtype=jnp.int32)
    g = jnp.concatenate([g.astype(jnp.int32), gpad + cyc % CH])
    s = jnp.concatenate([s.astype(jnp.int32), sbase + cyc % swin])
    iv = jnp.stack([g.reshape(NW, K, CH), s.reshape(NW, K, CH)], axis=2)
    lane = jnp.arange(CH, dtype=jnp.int32)
    tail = jnp.stack(
        [jnp.broadcast_to(gpad + lane % CH, (NW, 2, CH)),
         jnp.broadcast_to(sbase + lane % swin, (NW, 2, CH))], axis=2)
    return jnp.concatenate([iv, tail], axis=1).reshape(
        NW, (K + 2) // 2, 2, 2, CH)


def kernel(X, edge_index, he_nodes, he_edges, W, b):
    Xp = jnp.pad(X, ((0, R_G - N), (0, 0)))
    Wz = jnp.pad(W.T, ((0, 0), (0, WR - D)))
    b144 = jnp.zeros((8, WR), jnp.float32).at[0, :D].set(b).at[0, D].set(1.0)

    H = _tc_matmul(Xp, Wz, b144)

    iv = _pad_pairs(edge_index[0], edge_index[1], K_G, N, R_G - CH, R_G)
    pg = _sc_scatter(H, iv, R=R_G, K=K_G)

    ie = _pad_pairs(he_nodes, he_edges, K_E, N, NE, R_E)
    pe = _sc_scatter(H, ie, R=R_E, K=K_E)

    Yp = _tc_finalize_y(pe)

    ih = _pad_pairs(he_edges, he_nodes, K_E, R_E - CH, R_G - CH, R_G)
    ph = _sc_scatter(Yp, ih, R=R_G, K=K_E)

    out = _tc_combine(pg, ph)
    return out[:N]
